# baseline (speedup 1.0000x reference)
import functools

import jax
import jax.numpy as jnp
from jax import lax
from jax.experimental import pallas as pl
from jax.experimental.pallas import tpu as pltpu
from jax.experimental.pallas import tpu_sc as plsc

CTX = 200
PATH = 20
EMBED = 64
CTX_PAD = 256
PATH_PAD = 32

_mesh = plsc.ScalarSubcoreMesh(axis_name="c")


@functools.partial(
    pl.kernel,
    out_type=(
        jax.ShapeDtypeStruct((CTX_PAD, EMBED), jnp.float32),
        jax.ShapeDtypeStruct((PATH_PAD, EMBED), jnp.float32),
    ),
    mesh=_mesh,
    scratch_types=[
        pltpu.SMEM((CTX_PAD,), jnp.int32),
        pltpu.SMEM((PATH_PAD,), jnp.int32),
        pltpu.SemaphoreType.DMA,
    ],
)
def _sc_gather(ctx_idx_hbm, path_idx_hbm, ctx_table_hbm, node_table_hbm,
               crows_hbm, nrows_hbm, idx_s, pidx_s, sem):
    nw = _mesh.num_cores
    wid = lax.axis_index("c")

    pltpu.sync_copy(ctx_idx_hbm, idx_s)
    pltpu.sync_copy(path_idx_hbm, pidx_s)

    cpw = CTX // nw

    def issue_ctx(i, n):
        row = wid * cpw + i
        pltpu.async_copy(ctx_table_hbm.at[pl.ds(idx_s[row], 1)],
                         crows_hbm.at[pl.ds(row, 1)], sem)
        return n + 1

    ndma = lax.fori_loop(0, cpw, issue_ctx, 0)

    ppw = PATH_PAD // nw

    def issue_node(i, n):
        row = wid * ppw + i
        pltpu.async_copy(node_table_hbm.at[pl.ds(pidx_s[row], 1)],
                         nrows_hbm.at[pl.ds(row, 1)], sem)
        return n + 1

    ndma = lax.fori_loop(0, ppw, issue_node, ndma)

    def drain(i, _):
        pltpu.make_async_copy(ctx_table_hbm.at[pl.ds(0, 1)],
                              crows_hbm.at[pl.ds(0, 1)], sem).wait()
        return 0

    lax.fori_loop(0, ndma, drain, 0)


def _loss_body(crows_ref, n_ref, bits_ref, o_ref):
    h = (jnp.sum(crows_ref[...][:CTX], axis=0, keepdims=True)
         * (1.0 / CTX))
    n = n_ref[...]
    b = bits_ref[...]
    t = jnp.sum(n * h, axis=1, keepdims=True)
    s = jax.nn.sigmoid(t)
    eps = 1e-9
    per = -b * jnp.log(s + eps) - (1.0 - b) * jnp.log(1.0 - s + eps)
    row = lax.broadcasted_iota(jnp.int32, (PATH_PAD, 1), 0)
    per = jnp.where(row < PATH, per, 0.0)
    o_ref[0, 0] = jnp.sum(per)


_loss_call = pl.pallas_call(
    _loss_body,
    out_shape=jax.ShapeDtypeStruct((1, 1), jnp.float32),
    out_specs=pl.BlockSpec(memory_space=pltpu.SMEM),
)


def kernel(context_idx, path_indices, code_bits, context_table, node_table):
    ctx = jnp.asarray(context_idx, jnp.int32)
    pidx = jnp.asarray(path_indices, jnp.int32)
    ctx_pad = jnp.full((CTX_PAD,), ctx[CTX - 1], jnp.int32).at[:CTX].set(ctx)
    path_pad = jnp.zeros((PATH_PAD,), jnp.int32).at[:PATH].set(pidx)
    crows, nrows = _sc_gather(ctx_pad, path_pad, context_table, node_table)
    bits_col = (jnp.zeros((PATH_PAD, 1), jnp.float32)
                .at[:PATH, 0].set(code_bits.astype(jnp.float32)))
    out = _loss_call(crows, nrows, bits_col)
    return out[0, 0]


# speedup vs baseline: 1.0027x; 1.0027x over previous
import functools

import jax
import jax.numpy as jnp
from jax import lax
from jax.experimental import pallas as pl
from jax.experimental.pallas import tpu as pltpu
from jax.experimental.pallas import tpu_sc as plsc

CTX = 200
PATH = 20
EMBED = 64
LANES = 16
NVREG = EMBED // LANES
CTX_PAD = 272
PATH_PAD = 32
PATH_IDX_PAD = 48
NW = 32
CPW = 7

_mesh = plsc.VectorSubcoreMesh(core_axis_name="c", subcore_axis_name="s")


@functools.partial(
    pl.kernel,
    out_type=(
        jax.ShapeDtypeStruct((NW, EMBED), jnp.float32),
        jax.ShapeDtypeStruct((PATH_PAD, EMBED), jnp.float32),
    ),
    mesh=_mesh,
    scratch_types=[
        pltpu.VMEM((CTX_PAD,), jnp.int32),
        pltpu.VMEM((PATH_IDX_PAD,), jnp.int32),
        pltpu.VMEM((CPW, EMBED), jnp.float32),
        pltpu.VMEM((1, EMBED), jnp.float32),
        pltpu.VMEM((1, EMBED), jnp.float32),
        pltpu.SemaphoreType.DMA,
        pltpu.SemaphoreType.DMA,
    ],
)
def _sc_gather_pool(ctx_idx_hbm, path_idx_hbm, ctx_table_hbm, node_table_hbm,
                    part_hbm, nrows_hbm, idx_v, pidx_v, rows_v, acc_v, nrow_v,
                    sem, nsem):
    wid = lax.axis_index("s") * _mesh.num_cores + lax.axis_index("c")

    pltpu.sync_copy(ctx_idx_hbm, idx_v)
    pltpu.sync_copy(path_idx_hbm, pidx_v)

    nidx = pidx_v[pl.ds(wid, LANES)][0]
    cn = pltpu.async_copy(node_table_hbm.at[pl.ds(nidx, 1)], nrow_v, nsem)

    base = wid * CPW

    def issue(i, n):
        @pl.when(base + i < CTX)
        def _():
            cidx = idx_v[pl.ds(base + i, LANES)][0]
            pltpu.async_copy(ctx_table_hbm.at[pl.ds(cidx, 1)],
                             rows_v.at[pl.ds(i, 1)], sem)
        return n + jnp.where(base + i < CTX, 1, 0)

    ndma = lax.fori_loop(0, CPW, issue, 0)

    def drain(i, _):
        pltpu.make_async_copy(ctx_table_hbm.at[pl.ds(0, 1)],
                              rows_v.at[pl.ds(0, 1)], sem).wait()
        return 0

    lax.fori_loop(0, ndma, drain, 0)
    cn.wait()

    def pool(i, acc):
        take = base + i < CTX
        out = []
        for k in range(NVREG):
            v = rows_v[i, pl.ds(LANES * k, LANES)]
            out.append(acc[k] + jnp.where(take, v, 0.0))
        return tuple(out)

    acc = tuple(jnp.zeros((LANES,), jnp.float32) for _ in range(NVREG))
    acc = lax.fori_loop(0, CPW, pool, acc)
    for k in range(NVREG):
        acc_v[0, pl.ds(LANES * k, LANES)] = acc[k]

    pltpu.sync_copy(acc_v, part_hbm.at[pl.ds(wid, 1)])
    pltpu.sync_copy(nrow_v, nrows_hbm.at[pl.ds(wid, 1)])


def _loss_body(part_ref, n_ref, bits_ref, o_ref):
    h = (jnp.sum(part_ref[...], axis=0, keepdims=True) * (1.0 / CTX))
    n = n_ref[...]
    b = bits_ref[...]
    t = jnp.sum(n * h, axis=1, keepdims=True)
    s = jax.nn.sigmoid(t)
    eps = 1e-9
    per = -b * jnp.log(s + eps) - (1.0 - b) * jnp.log(1.0 - s + eps)
    row = lax.broadcasted_iota(jnp.int32, (PATH_PAD, 1), 0)
    per = jnp.where(row < PATH, per, 0.0)
    o_ref[0, 0] = jnp.sum(per)


_loss_call = pl.pallas_call(
    _loss_body,
    out_shape=jax.ShapeDtypeStruct((1, 1), jnp.float32),
    out_specs=pl.BlockSpec(memory_space=pltpu.SMEM),
)


def kernel(context_idx, path_indices, code_bits, context_table, node_table):
    ctx = jnp.asarray(context_idx, jnp.int32)
    pidx = jnp.asarray(path_indices, jnp.int32)
    ctx_pad = jnp.zeros((CTX_PAD,), jnp.int32).at[:CTX].set(ctx)
    path_pad = jnp.zeros((PATH_IDX_PAD,), jnp.int32).at[:PATH].set(pidx)
    part, nrows = _sc_gather_pool(ctx_pad, path_pad, context_table, node_table)
    bits_col = (jnp.zeros((PATH_PAD, 1), jnp.float32)
                .at[:PATH, 0].set(code_bits.astype(jnp.float32)))
    out = _loss_call(part, nrows, bits_col)
    return out[0, 0]
